# Initial kernel scaffold; baseline (speedup 1.0000x reference)
import jax
import jax.numpy as jnp
from jax import lax
from jax.experimental import pallas as pl
from jax.experimental.pallas import tpu as pltpu
from jax.experimental.pallas import tpu_sc as plsc

B = 4096
V = 1000
VP = 1024
D = 16
NW = 32
IPW = B // NW
LANES = 16
BLK = 512


def _sc_hist_body(idx_hbm, out_hbm, idx_v, hist_v):
    c = lax.axis_index("c")
    s = lax.axis_index("s")
    wid = s * 2 + c
    zeros16 = jnp.zeros((LANES,), jnp.float32)
    for i in range(VP // LANES):
        hist_v[pl.ds(i * LANES, LANES)] = zeros16
    pltpu.sync_copy(idx_hbm.at[pl.ds(wid * IPW, IPW)], idx_v)
    ones16 = jnp.ones((LANES,), jnp.float32)
    for ch in range(IPW // LANES):
        v = idx_v[pl.ds(ch * LANES, LANES)]
        plsc.addupdate_scatter(hist_v, [v], ones16)
    pltpu.sync_copy(hist_v, out_hbm.at[wid])


_sc_hist = pl.kernel(
    _sc_hist_body,
    mesh=plsc.VectorSubcoreMesh(core_axis_name="c", subcore_axis_name="s"),
    out_type=jax.ShapeDtypeStruct((NW, VP), jnp.float32),
    scratch_types=[
        pltpu.VMEM((IPW,), jnp.int32),
        pltpu.VMEM((VP,), jnp.float32),
    ],
)


def _loss_body(u_ref, e_ref, ic_ref, part_ref, idx_ref, o_ref):
    u = u_ref[...]
    e = e_ref[...]
    ic = ic_ref[...]
    cnt = jnp.sum(part_ref[...], axis=0, keepdims=True)
    idxb = idx_ref[...]
    s = lax.dot_general(u, e, (((1,), (1,)), ((), ())),
                        preferred_element_type=jnp.float32)
    logq = jnp.log(ic / jnp.sum(ic))
    t = s - logq
    live = cnt > 0.0
    tm = jnp.where(live, t, -jnp.inf)
    m = jnp.max(tm, axis=1, keepdims=True)
    se = jnp.sum(cnt * jnp.exp(tm - m), axis=1, keepdims=True)
    col = lax.broadcasted_iota(jnp.int32, (BLK, VP), 1)
    d = jnp.sum(jnp.where(col == idxb, t, 0.0), axis=1, keepdims=True)
    o_ref[...] = m + jnp.log(se) - d


def kernel(item_embeddings, user_vec, item_count, item_idx):
    idx = item_idx.reshape(B).astype(jnp.int32)
    part = _sc_hist(idx)
    e_pad = jnp.pad(item_embeddings, ((0, VP - V), (0, 0)))
    ic_pad = jnp.pad(item_count.reshape(1, V), ((0, 0), (0, VP - V)))
    return pl.pallas_call(
        _loss_body,
        grid=(B // BLK,),
        in_specs=[
            pl.BlockSpec((BLK, D), lambda i: (i, 0)),
            pl.BlockSpec((VP, D), lambda i: (0, 0)),
            pl.BlockSpec((1, VP), lambda i: (0, 0)),
            pl.BlockSpec((NW, VP), lambda i: (0, 0)),
            pl.BlockSpec((BLK, 1), lambda i: (i, 0)),
        ],
        out_specs=pl.BlockSpec((BLK, 1), lambda i: (i, 0)),
        out_shape=jax.ShapeDtypeStruct((B, 1), jnp.float32),
    )(user_vec, e_pad, ic_pad, part, idx.reshape(B, 1))


# speedup vs baseline: 3.0057x; 3.0057x over previous
import jax
import jax.numpy as jnp
from jax import lax
from jax.experimental import pallas as pl
from jax.experimental.pallas import tpu as pltpu
from jax.experimental.pallas import tpu_sc as plsc

B = 4096
V = 1000
VP = 1024
D = 16
NW = 32
IPW = B // NW
LANES = 16
BLK = 512


def _sc_hist_body(idx_hbm, out_hbm, idx_v, hist_v):
    c = lax.axis_index("c")
    s = lax.axis_index("s")
    wid = s * 2 + c
    zeros16 = jnp.zeros((LANES,), jnp.float32)
    for i in range(VP // LANES):
        hist_v[pl.ds(i * LANES, LANES)] = zeros16
    pltpu.sync_copy(idx_hbm.at[pl.ds(wid * IPW, IPW)], idx_v)
    ones16 = jnp.ones((LANES,), jnp.float32)
    for ch in range(IPW // LANES):
        v = idx_v[pl.ds(ch * LANES, LANES)]
        plsc.addupdate_scatter(hist_v, [v], ones16)
    pltpu.sync_copy(hist_v, out_hbm.at[wid])


def _sc_hist(idx):
    return pl.kernel(
        _sc_hist_body,
        mesh=plsc.VectorSubcoreMesh(core_axis_name="c", subcore_axis_name="s"),
        out_type=jax.ShapeDtypeStruct((NW, VP), jnp.float32),
        scratch_types=[
            pltpu.VMEM((IPW,), jnp.int32),
            pltpu.VMEM((VP,), jnp.float32),
        ],
        compiler_params=pltpu.CompilerParams(needs_layout_passes=False),
    )(idx)


def _loss_body(u_ref, e_ref, ic_ref, part_ref, idx_ref, o_ref):
    u = u_ref[...]
    e = e_ref[...]
    ic = ic_ref[...]
    cnt = jnp.sum(part_ref[...], axis=0, keepdims=True)
    idxb = idx_ref[...]
    s = lax.dot_general(u, e, (((1,), (1,)), ((), ())),
                        preferred_element_type=jnp.float32)
    logq = jnp.log(ic / jnp.sum(ic))
    t = s - logq
    live = cnt > 0.0
    tm = jnp.where(live, t, -jnp.inf)
    m = jnp.max(tm, axis=1, keepdims=True)
    se = jnp.sum(cnt * jnp.exp(tm - m), axis=1, keepdims=True)
    col = lax.broadcasted_iota(jnp.int32, (BLK, VP), 1)
    d = jnp.sum(jnp.where(col == idxb, t, 0.0), axis=1, keepdims=True)
    o_ref[...] = m + jnp.log(se) - d


def kernel(item_embeddings, user_vec, item_count, item_idx):
    idx = item_idx.reshape(B).astype(jnp.int32)
    part = _sc_hist(idx)
    e_pad = jnp.pad(item_embeddings, ((0, VP - V), (0, 0)))
    ic_pad = jnp.pad(item_count.reshape(1, V), ((0, 0), (0, VP - V)))
    return pl.pallas_call(
        _loss_body,
        grid=(B // BLK,),
        in_specs=[
            pl.BlockSpec((BLK, D), lambda i: (i, 0)),
            pl.BlockSpec((VP, D), lambda i: (0, 0)),
            pl.BlockSpec((1, VP), lambda i: (0, 0)),
            pl.BlockSpec((NW, VP), lambda i: (0, 0)),
            pl.BlockSpec((BLK, 1), lambda i: (i, 0)),
        ],
        out_specs=pl.BlockSpec((BLK, 1), lambda i: (i, 0)),
        out_shape=jax.ShapeDtypeStruct((B, 1), jnp.float32),
    )(user_vec, e_pad, ic_pad, part, idx.reshape(B, 1))
